# Initial kernel scaffold; baseline (speedup 1.0000x reference)
import jax
import jax.numpy as jnp
from jax.experimental import pallas as pl


def kernel(feature, edge_index, emb_W, emb_b, se_W1, se_b1, se_W2, se_b2, fc_W, attn_l, attn_r, gat_bias, proj_W, proj_b, graph_W, w_param):
    raise NotImplementedError("write your pallas kernel here")


# speedup vs baseline: 112.9676x; 112.9676x over previous
import functools

import jax
import jax.numpy as jnp
from jax import lax
from jax.experimental import pallas as pl
from jax.experimental.pallas import tpu as pltpu
from jax.experimental.pallas import tpu_sc as plsc

B, C, L = 512, 64, 96
EMB = 16
H, F = 4, 4
HF = H * F
GH = 4
N = B * C
E = 1048576

NC, NS = 2, 16
NW = NC * NS
EPW = E // NW
K = 128
SRCW = 32
ERW = 16
ACCW = 24
NPT = N // NS


PB = 128
PN = PB * C


def _prep_body(feature, emb_W, emb_b, se_W1, se_b1, se_W2, se_b2, fc_W,
               AL, AR, src_tab, er_tab, m_ref):
    i = pl.program_id(0)
    x = feature[...].reshape(PN, L) @ emb_W[...] + emb_b[...]
    x3 = x.reshape(PB, C, EMB)
    s = jnp.mean(x3, axis=1)
    a = jax.nn.relu(s @ se_W1[...] + se_b1[...])
    g = jax.nn.sigmoid(a @ se_W2[...] + se_b2[...])
    embed = (x3 * g[:, None, :]).reshape(PN, EMB)
    feat = embed @ fc_W[...]
    el = feat @ AL[...]
    er = feat @ AR[...]
    mel = jnp.broadcast_to(jnp.max(el, axis=0)[:, None], (H, 16))
    mer = jnp.broadcast_to(jnp.max(er, axis=0)[:, None], (H, 16))
    mb = jnp.concatenate([mel, mer], axis=0)

    @pl.when(i == 0)
    def _():
        m_ref[...] = mb

    @pl.when(i > 0)
    def _():
        m_ref[...] = jnp.maximum(m_ref[...], mb)

    @pl.when(i == B // PB - 1)
    def _():
        m4 = m_ref[0:4, :] + m_ref[4:8, :]
        m_ref[0:4, :] = jnp.maximum(m4, 0.2 * m4)

    pad12 = jnp.zeros((PN, 12), jnp.float32)
    src_tab[...] = jnp.concatenate([el, feat, pad12], axis=1)
    er_tab[...] = jnp.concatenate([er, pad12], axis=1)


def _prep(feature, emb_W, emb_b, se_W1, se_b1, se_W2, se_b2, fc_W, AL, AR):
    grid = B // PB
    small = lambda shape: pl.BlockSpec(shape, lambda i: (0,) * len(shape))
    return pl.pallas_call(
        _prep_body,
        grid=(grid,),
        in_specs=[
            pl.BlockSpec((PB, C, L), lambda i: (i, 0, 0)),
            small((L, EMB)),
            small((1, EMB)),
            small((EMB, 4)),
            small((1, 4)),
            small((4, EMB)),
            small((1, EMB)),
            small((EMB, HF)),
            small((HF, H)),
            small((HF, H)),
        ],
        out_specs=[
            pl.BlockSpec((PN, SRCW), lambda i: (i, 0)),
            pl.BlockSpec((PN, ERW), lambda i: (i, 0)),
            pl.BlockSpec((8, 16), lambda i: (0, 0)),
        ],
        out_shape=[
            jax.ShapeDtypeStruct((N, SRCW), jnp.float32),
            jax.ShapeDtypeStruct((N, ERW), jnp.float32),
            jax.ShapeDtypeStruct((8, 16), jnp.float32),
        ],
    )(feature, emb_W, emb_b, se_W1, se_b1, se_W2, se_b2, fc_W, AL, AR)


def _edge_body(src_hbm, dst_hbm, src_tab, er_tab, m_hbm, zeros_hbm, out_hbm,
               sidx, didx, srcrows, errows, outrows, m_v, acc, sem1, sem2):
    cid = lax.axis_index("c")
    sid = lax.axis_index("s")
    wid = cid * NS + sid

    pltpu.sync_copy(m_hbm.at[pl.ds(0, H)], m_v)
    pltpu.sync_copy(zeros_hbm, acc.at[pl.ds(sid * NPT, NPT)])
    plsc.subcore_barrier()

    ii = lax.iota(jnp.int32, 16)
    zero16 = jnp.zeros((16,), jnp.float32)
    for j in range(K // 16):
        rows = ii + j * 16
        for c in range(4 + HF, ACCW):
            plsc.store_scatter(outrows, [rows, jnp.full((16,), c, jnp.int32)],
                               zero16)

    ebase = wid * EPW

    def chunk(ci, carry):
        off = ebase + ci * K
        pltpu.sync_copy(src_hbm.at[pl.ds(off, K)], sidx)
        pltpu.sync_copy(dst_hbm.at[pl.ds(off, K)], didx)
        d1 = pltpu.async_copy(src_tab.at[sidx], srcrows, sem1)
        d2 = pltpu.async_copy(er_tab.at[didx], errows, sem2)
        d1.wait()
        d2.wait()
        for j in range(K // 16):
            rows = ii + j * 16
            for h in range(H):
                ch = jnp.full((16,), h, jnp.int32)
                e = (plsc.load_gather(srcrows, [rows, ch]) +
                     plsc.load_gather(errows, [rows, ch]))
                e = jnp.maximum(e, 0.2 * e)
                ee = jnp.exp(e - m_v[h])
                plsc.store_scatter(outrows, [rows, ch], ee)
                for f in range(F):
                    c = 4 + h * F + f
                    cc = jnp.full((16,), c, jnp.int32)
                    fv = plsc.load_gather(srcrows, [rows, cc])
                    plsc.store_scatter(outrows, [rows, cc], ee * fv)
        pltpu.sync_copy(outrows, acc.at[didx], add=True)
        return carry

    lax.fori_loop(0, EPW // K, chunk, 0)

    plsc.subcore_barrier()
    pltpu.sync_copy(acc.at[pl.ds(sid * NPT, NPT)],
                    out_hbm.at[pl.ds(wid * NPT, NPT)])


def _edge(src, dst, src_tab, er_tab, m, zeros):
    mesh = plsc.VectorSubcoreMesh(core_axis_name="c", subcore_axis_name="s")
    f = functools.partial(
        pl.kernel,
        out_type=jax.ShapeDtypeStruct((NC * N, ACCW), jnp.float32),
        mesh=mesh,
        scratch_types=[
            pltpu.VMEM((K,), jnp.int32),
            pltpu.VMEM((K,), jnp.int32),
            pltpu.VMEM((K, SRCW), jnp.float32),
            pltpu.VMEM((K, ERW), jnp.float32),
            pltpu.VMEM((K, ACCW), jnp.float32),
            pltpu.VMEM((H, 16), jnp.float32),
            pltpu.VMEM_SHARED((N, ACCW), jnp.float32),
            pltpu.SemaphoreType.DMA,
            pltpu.SemaphoreType.DMA,
        ],
        compiler_params=pltpu.CompilerParams(needs_layout_passes=False,
                                             use_tc_tiling_on_sc=False),
    )(_edge_body)
    return f(src, dst, src_tab, er_tab, m, zeros)


GB = 64
GN = GB * C


def _post_body(p0, p1, gat_bias, proj_W, proj_b, gw1, gw2, w_ref, out_ref):
    accb = p0[...] + p1[...]
    den = accb[:, 0:4]
    msg = accb[:, 4:4 + HF]
    hrow = lax.broadcasted_iota(jnp.int32, (H, HF), 0)
    hcol = lax.broadcasted_iota(jnp.int32, (H, HF), 1) // F
    R = jnp.where(hrow == hcol, 1.0, 0.0)
    den16 = den @ R
    rst = jnp.where(den16 != 0.0, msg / den16, 0.0) + gat_bias[...]
    encode = rst.reshape(GB, C, HF)
    graph1 = lax.dot_general(encode, encode,
                             (((2,), (2,)), ((0,), (0,))))
    h = lax.dot_general(encode, proj_W[...],
                        (((2,), (0,)), ((), ()))) + proj_b[...][None]
    u = lax.dot_general(h, gw1[...], (((2,), (0,)), ((), ())))
    v = lax.dot_general(h, gw2[...], (((2,), (0,)), ((), ())))
    graph2 = u + v[:, :, 0][:, None, :]
    w = w_ref[...][None]
    out_ref[...] = w * graph1 + (1.0 - w) * graph2


def _post(partial, gat_bias, proj_W, proj_b, gw1, gw2, w_param):
    grid = B // GB
    small = lambda shape: pl.BlockSpec(shape, lambda i: (0,) * len(shape))
    return pl.pallas_call(
        _post_body,
        grid=(grid,),
        in_specs=[
            pl.BlockSpec((GN, ACCW), lambda i: (i, 0)),
            pl.BlockSpec((GN, ACCW), lambda i: (i + grid, 0)),
            small((1, HF)),
            small((EMB, GH)),
            small((1, GH)),
            small((GH, 1)),
            small((GH, 1)),
            small((C, C)),
        ],
        out_specs=pl.BlockSpec((GB, C, C), lambda i: (i, 0, 0)),
        out_shape=jax.ShapeDtypeStruct((B, C, C), jnp.float32),
    )(partial, partial, gat_bias, proj_W, proj_b, gw1, gw2, w_param)


def kernel(feature, edge_index, emb_W, emb_b, se_W1, se_b1, se_W2, se_b2,
           fc_W, attn_l, attn_r, gat_bias, proj_W, proj_b, graph_W, w_param):
    src = edge_index[0]
    dst = edge_index[1]
    eye = jnp.eye(H, dtype=jnp.float32)
    AL = (eye[:, None, :] * attn_l[:, :, None]).reshape(HF, H)
    AR = (eye[:, None, :] * attn_r[:, :, None]).reshape(HF, H)

    src_tab, er_tab, m = _prep(feature, emb_W, emb_b.reshape(1, EMB),
                               se_W1, se_b1.reshape(1, 4), se_W2,
                               se_b2.reshape(1, EMB), fc_W, AL, AR)
    zeros = jnp.zeros((NPT, ACCW), jnp.float32)
    partial = _edge(src, dst, src_tab, er_tab, m, zeros)
    return _post(partial, gat_bias.reshape(1, HF), proj_W,
                 proj_b.reshape(1, GH), graph_W[:GH], graph_W[GH:], w_param)


# speedup vs baseline: 125.8046x; 1.1136x over previous
import functools

import jax
import jax.numpy as jnp
from jax import lax
from jax.experimental import pallas as pl
from jax.experimental.pallas import tpu as pltpu
from jax.experimental.pallas import tpu_sc as plsc

B, C, L = 512, 64, 96
EMB = 16
H, F = 4, 4
HF = H * F
GH = 4
N = B * C
E = 1048576

NC, NS = 2, 16
NW = NC * NS
EPW = E // NW
K = 128
SRCW = 32
ERW = 16
ACCW = 24
NPT = N // NS


PB = 128
PN = PB * C


def _prep_body(feature, emb_W, emb_b, se_W1, se_b1, se_W2, se_b2, fc_W,
               AL, AR, src_tab, er_tab, m_ref):
    i = pl.program_id(0)
    x = feature[...].reshape(PN, L) @ emb_W[...] + emb_b[...]
    x3 = x.reshape(PB, C, EMB)
    s = jnp.mean(x3, axis=1)
    a = jax.nn.relu(s @ se_W1[...] + se_b1[...])
    g = jax.nn.sigmoid(a @ se_W2[...] + se_b2[...])
    embed = (x3 * g[:, None, :]).reshape(PN, EMB)
    feat = embed @ fc_W[...]
    el = feat @ AL[...]
    er = feat @ AR[...]
    mel = jnp.broadcast_to(jnp.max(el, axis=0)[:, None], (H, 16))
    mer = jnp.broadcast_to(jnp.max(er, axis=0)[:, None], (H, 16))
    mb = jnp.concatenate([mel, mer], axis=0)

    @pl.when(i == 0)
    def _():
        m_ref[...] = mb

    @pl.when(i > 0)
    def _():
        m_ref[...] = jnp.maximum(m_ref[...], mb)

    @pl.when(i == B // PB - 1)
    def _():
        m4 = m_ref[0:4, :] + m_ref[4:8, :]
        m_ref[0:4, :] = jnp.maximum(m4, 0.2 * m4)

    pad12 = jnp.zeros((PN, 12), jnp.float32)
    src_tab[...] = jnp.concatenate([el, feat, pad12], axis=1)
    er_tab[...] = jnp.concatenate([er, pad12], axis=1)


def _prep(feature, emb_W, emb_b, se_W1, se_b1, se_W2, se_b2, fc_W, AL, AR):
    grid = B // PB
    small = lambda shape: pl.BlockSpec(shape, lambda i: (0,) * len(shape))
    return pl.pallas_call(
        _prep_body,
        grid=(grid,),
        in_specs=[
            pl.BlockSpec((PB, C, L), lambda i: (i, 0, 0)),
            small((L, EMB)),
            small((1, EMB)),
            small((EMB, 4)),
            small((1, 4)),
            small((4, EMB)),
            small((1, EMB)),
            small((EMB, HF)),
            small((HF, H)),
            small((HF, H)),
        ],
        out_specs=[
            pl.BlockSpec((PN, SRCW), lambda i: (i, 0)),
            pl.BlockSpec((PN, ERW), lambda i: (i, 0)),
            pl.BlockSpec((8, 16), lambda i: (0, 0)),
        ],
        out_shape=[
            jax.ShapeDtypeStruct((N, SRCW), jnp.float32),
            jax.ShapeDtypeStruct((N, ERW), jnp.float32),
            jax.ShapeDtypeStruct((8, 16), jnp.float32),
        ],
    )(feature, emb_W, emb_b, se_W1, se_b1, se_W2, se_b2, fc_W, AL, AR)


def _edge_body(src_hbm, dst_hbm, src_tab, er_tab, m_hbm, zeros_hbm, out_hbm,
               sidx0, didx0, sidx1, didx1, dsc0, dsc1, sr0, erb0, sr1, erb1,
               ob0, ob1, m_v, acc, sg0, sg1, ss0, ss1):
    cid = lax.axis_index("c")
    sid = lax.axis_index("s")
    wid = cid * NS + sid

    sidx = (sidx0, sidx1)
    didx = (didx0, didx1)
    dsc = (dsc0, dsc1)
    srb = (sr0, sr1)
    erb = (erb0, erb1)
    ob = (ob0, ob1)
    sg = (sg0, sg1)
    ss = (ss0, ss1)

    pltpu.sync_copy(m_hbm.at[pl.ds(0, H)], m_v)
    pltpu.sync_copy(zeros_hbm, acc.at[pl.ds(sid * NPT, NPT)])
    plsc.subcore_barrier()

    ii = lax.iota(jnp.int32, 16)
    zero16 = jnp.zeros((16,), jnp.float32)
    for buf in ob:
        for j in range(K // 16):
            rows = ii + j * 16
            for c in range(4 + HF, ACCW):
                plsc.store_scatter(buf, [rows, jnp.full((16,), c, jnp.int32)],
                                   zero16)

    ebase = wid * EPW
    NCH = EPW // K

    def load_idx(ci, b):
        off = ebase + ci * K
        pltpu.sync_copy(src_hbm.at[pl.ds(off, K)], sidx[b])
        pltpu.sync_copy(dst_hbm.at[pl.ds(off, K)], didx[b])

    def issue_gathers(b):
        pltpu.async_copy(src_tab.at[sidx[b]], srb[b], sg[b])
        pltpu.async_copy(er_tab.at[didx[b]], erb[b], sg[b])

    def wait_gathers(b):
        pltpu.make_async_copy(src_tab.at[sidx[b]], srb[b], sg[b]).wait()
        pltpu.make_async_copy(er_tab.at[didx[b]], erb[b], sg[b]).wait()

    def drain_scatter(b):
        pltpu.make_async_copy(ob[b], acc.at[dsc[b]], ss[b]).wait()

    load_idx(0, 0)
    issue_gathers(0)

    def outer(g, carry):
        for b in range(2):
            ci = g * 2 + b
            wait_gathers(b)

            @pl.when(ci + 1 < NCH)
            def _():
                load_idx(ci + 1, 1 - b)
                issue_gathers(1 - b)

            @pl.when(ci >= 2)
            def _():
                drain_scatter(b)

            for j in range(K // 16):
                rows = ii + j * 16
                for h in range(H):
                    ch = jnp.full((16,), h, jnp.int32)
                    e = (plsc.load_gather(srb[b], [rows, ch]) +
                         plsc.load_gather(erb[b], [rows, ch]))
                    e = jnp.maximum(e, 0.2 * e)
                    ee = jnp.exp(e - m_v[h])
                    plsc.store_scatter(ob[b], [rows, ch], ee)
                    for f in range(F):
                        c = 4 + h * F + f
                        cc = jnp.full((16,), c, jnp.int32)
                        fv = plsc.load_gather(srb[b], [rows, cc])
                        plsc.store_scatter(ob[b], [rows, cc], ee * fv)
            for j in range(K // 16):
                dsc[b][pl.ds(j * 16, 16)] = didx[b][pl.ds(j * 16, 16)]
            pltpu.async_copy(ob[b], acc.at[dsc[b]], ss[b], add=True)
        return carry

    lax.fori_loop(0, NCH // 2, outer, 0)
    drain_scatter(0)
    drain_scatter(1)

    plsc.subcore_barrier()
    pltpu.sync_copy(acc.at[pl.ds(sid * NPT, NPT)],
                    out_hbm.at[pl.ds(wid * NPT, NPT)])


def _edge(src, dst, src_tab, er_tab, m, zeros):
    mesh = plsc.VectorSubcoreMesh(core_axis_name="c", subcore_axis_name="s")
    f = functools.partial(
        pl.kernel,
        out_type=jax.ShapeDtypeStruct((NC * N, ACCW), jnp.float32),
        mesh=mesh,
        scratch_types=[
            pltpu.VMEM((K,), jnp.int32),
            pltpu.VMEM((K,), jnp.int32),
            pltpu.VMEM((K,), jnp.int32),
            pltpu.VMEM((K,), jnp.int32),
            pltpu.VMEM((K,), jnp.int32),
            pltpu.VMEM((K,), jnp.int32),
            pltpu.VMEM((K, SRCW), jnp.float32),
            pltpu.VMEM((K, ERW), jnp.float32),
            pltpu.VMEM((K, SRCW), jnp.float32),
            pltpu.VMEM((K, ERW), jnp.float32),
            pltpu.VMEM((K, ACCW), jnp.float32),
            pltpu.VMEM((K, ACCW), jnp.float32),
            pltpu.VMEM((H, 16), jnp.float32),
            pltpu.VMEM_SHARED((N, ACCW), jnp.float32),
            pltpu.SemaphoreType.DMA,
            pltpu.SemaphoreType.DMA,
            pltpu.SemaphoreType.DMA,
            pltpu.SemaphoreType.DMA,
        ],
        compiler_params=pltpu.CompilerParams(needs_layout_passes=False,
                                             use_tc_tiling_on_sc=False),
    )(_edge_body)
    return f(src, dst, src_tab, er_tab, m, zeros)


GB = 64
GN = GB * C


def _post_body(p0, p1, gat_bias, proj_W, proj_b, gw1, gw2, w_ref, out_ref):
    accb = p0[...] + p1[...]
    den = accb[:, 0:4]
    msg = accb[:, 4:4 + HF]
    hrow = lax.broadcasted_iota(jnp.int32, (H, HF), 0)
    hcol = lax.broadcasted_iota(jnp.int32, (H, HF), 1) // F
    R = jnp.where(hrow == hcol, 1.0, 0.0)
    den16 = den @ R
    rst = jnp.where(den16 != 0.0, msg / den16, 0.0) + gat_bias[...]
    encode = rst.reshape(GB, C, HF)
    graph1 = lax.dot_general(encode, encode,
                             (((2,), (2,)), ((0,), (0,))))
    h = lax.dot_general(encode, proj_W[...],
                        (((2,), (0,)), ((), ()))) + proj_b[...][None]
    u = lax.dot_general(h, gw1[...], (((2,), (0,)), ((), ())))
    v = lax.dot_general(h, gw2[...], (((2,), (0,)), ((), ())))
    graph2 = u + v[:, :, 0][:, None, :]
    w = w_ref[...][None]
    out_ref[...] = w * graph1 + (1.0 - w) * graph2


def _post(partial, gat_bias, proj_W, proj_b, gw1, gw2, w_param):
    grid = B // GB
    small = lambda shape: pl.BlockSpec(shape, lambda i: (0,) * len(shape))
    return pl.pallas_call(
        _post_body,
        grid=(grid,),
        in_specs=[
            pl.BlockSpec((GN, ACCW), lambda i: (i, 0)),
            pl.BlockSpec((GN, ACCW), lambda i: (i + grid, 0)),
            small((1, HF)),
            small((EMB, GH)),
            small((1, GH)),
            small((GH, 1)),
            small((GH, 1)),
            small((C, C)),
        ],
        out_specs=pl.BlockSpec((GB, C, C), lambda i: (i, 0, 0)),
        out_shape=jax.ShapeDtypeStruct((B, C, C), jnp.float32),
    )(partial, partial, gat_bias, proj_W, proj_b, gw1, gw2, w_param)


def kernel(feature, edge_index, emb_W, emb_b, se_W1, se_b1, se_W2, se_b2,
           fc_W, attn_l, attn_r, gat_bias, proj_W, proj_b, graph_W, w_param):
    src = edge_index[0]
    dst = edge_index[1]
    eye = jnp.eye(H, dtype=jnp.float32)
    AL = (eye[:, None, :] * attn_l[:, :, None]).reshape(HF, H)
    AR = (eye[:, None, :] * attn_r[:, :, None]).reshape(HF, H)

    src_tab, er_tab, m = _prep(feature, emb_W, emb_b.reshape(1, EMB),
                               se_W1, se_b1.reshape(1, 4), se_W2,
                               se_b2.reshape(1, EMB), fc_W, AL, AR)
    zeros = jnp.zeros((NPT, ACCW), jnp.float32)
    partial = _edge(src, dst, src_tab, er_tab, m, zeros)
    return _post(partial, gat_bias.reshape(1, HF), proj_W,
                 proj_b.reshape(1, GH), graph_W[:GH], graph_W[GH:], w_param)


# speedup vs baseline: 158.1239x; 1.2569x over previous
import functools

import jax
import jax.numpy as jnp
from jax import lax
from jax.experimental import pallas as pl
from jax.experimental.pallas import tpu as pltpu
from jax.experimental.pallas import tpu_sc as plsc

B, C, L = 512, 64, 96
EMB = 16
H, F = 4, 4
HF = H * F
GH = 4
N = B * C
E = 1048576

NC, NS = 2, 16
NW = NC * NS
EPW = E // NW
K = 128
SRCW = 32
ERW = 16
ACCW = 24
NPT = N // NS


PB = 128
PN = PB * C


def _prep_body(feature, emb_W, emb_b, se_W1, se_b1, se_W2, se_b2, fc_W,
               AL, AR, src_tab, er_tab, m_ref):
    i = pl.program_id(0)
    x = feature[...].reshape(PN, L) @ emb_W[...] + emb_b[...]
    x3 = x.reshape(PB, C, EMB)
    s = jnp.mean(x3, axis=1)
    a = jax.nn.relu(s @ se_W1[...] + se_b1[...])
    g = jax.nn.sigmoid(a @ se_W2[...] + se_b2[...])
    embed = (x3 * g[:, None, :]).reshape(PN, EMB)
    feat = embed @ fc_W[...]
    el = feat @ AL[...]
    er = feat @ AR[...]
    mel = jnp.broadcast_to(jnp.max(el, axis=0)[:, None], (H, 16))
    mer = jnp.broadcast_to(jnp.max(er, axis=0)[:, None], (H, 16))
    mb = jnp.concatenate([mel, mer], axis=0)

    @pl.when(i == 0)
    def _():
        m_ref[...] = mb

    @pl.when(i > 0)
    def _():
        m_ref[...] = jnp.maximum(m_ref[...], mb)

    @pl.when(i == B // PB - 1)
    def _():
        m4 = m_ref[0:4, :] + m_ref[4:8, :]
        m_ref[0:4, :] = jnp.maximum(m4, 0.2 * m4)

    pad12 = jnp.zeros((PN, 12), jnp.float32)
    src_tab[...] = jnp.concatenate([el, feat, pad12], axis=1)
    er_tab[...] = jnp.concatenate([er, pad12], axis=1)


def _prep(feature, emb_W, emb_b, se_W1, se_b1, se_W2, se_b2, fc_W, AL, AR):
    grid = B // PB
    small = lambda shape: pl.BlockSpec(shape, lambda i: (0,) * len(shape))
    return pl.pallas_call(
        _prep_body,
        grid=(grid,),
        in_specs=[
            pl.BlockSpec((PB, C, L), lambda i: (i, 0, 0)),
            small((L, EMB)),
            small((1, EMB)),
            small((EMB, 4)),
            small((1, 4)),
            small((4, EMB)),
            small((1, EMB)),
            small((EMB, HF)),
            small((HF, H)),
            small((HF, H)),
        ],
        out_specs=[
            pl.BlockSpec((PN, SRCW), lambda i: (i, 0)),
            pl.BlockSpec((PN, ERW), lambda i: (i, 0)),
            pl.BlockSpec((8, 16), lambda i: (0, 0)),
        ],
        out_shape=[
            jax.ShapeDtypeStruct((N, SRCW), jnp.float32),
            jax.ShapeDtypeStruct((N, ERW), jnp.float32),
            jax.ShapeDtypeStruct((8, 16), jnp.float32),
        ],
    )(feature, emb_W, emb_b, se_W1, se_b1, se_W2, se_b2, fc_W, AL, AR)


CPB = 16
RPW = EPW // K
NB = RPW // CPB


def _edge_body(src_hbm, dst_hbm, src_tab, er_tab, m_hbm, zeros_hbm, out_hbm,
               sblk, dblk, sr0, erb0, sr1, erb1,
               ob0, ob1, m_v, acc, si, sg0, sg1, ss0, ss1):
    cid = lax.axis_index("c")
    sid = lax.axis_index("s")
    wid = cid * NS + sid

    srb = (sr0, sr1)
    erb = (erb0, erb1)
    ob = (ob0, ob1)
    sg = (sg0, sg1)
    ss = (ss0, ss1)

    pltpu.sync_copy(m_hbm.at[pl.ds(0, H)], m_v)
    pltpu.sync_copy(zeros_hbm, acc.at[pl.ds(sid * NPT, NPT)])
    plsc.subcore_barrier()

    ii = lax.iota(jnp.int32, 16)
    zero16 = jnp.zeros((16,), jnp.float32)
    for buf in ob:
        for j in range(K // 16):
            rows = ii + j * 16
            for c in range(4 + HF, ACCW):
                plsc.store_scatter(buf, [rows, jnp.full((16,), c, jnp.int32)],
                                   zero16)

    rbase = wid * RPW

    def load_block(t):
        half = (t % 2) * CPB
        pltpu.async_copy(src_hbm.at[pl.ds(rbase + t * CPB, CPB)],
                         sblk.at[pl.ds(half, CPB)], si)
        pltpu.async_copy(dst_hbm.at[pl.ds(rbase + t * CPB, CPB)],
                         dblk.at[pl.ds(half, CPB)], si)

    def wait_block():
        pltpu.make_async_copy(src_hbm.at[pl.ds(0, CPB)],
                              sblk.at[pl.ds(0, CPB)], si).wait()
        pltpu.make_async_copy(dst_hbm.at[pl.ds(0, CPB)],
                              dblk.at[pl.ds(0, CPB)], si).wait()

    def issue_gathers(ci, b):
        r = ci % (2 * CPB)
        pltpu.async_copy(src_tab.at[sblk.at[r]], srb[b], sg[b])
        pltpu.async_copy(er_tab.at[dblk.at[r]], erb[b], sg[b])

    def wait_gathers(b):
        pltpu.make_async_copy(src_tab.at[sblk.at[0]], srb[b], sg[b]).wait()
        pltpu.make_async_copy(er_tab.at[dblk.at[0]], erb[b], sg[b]).wait()

    def drain_scatter(b):
        pltpu.make_async_copy(ob[b], acc.at[dblk.at[0]], ss[b]).wait()

    load_block(0)
    wait_block()
    issue_gathers(0, 0)

    def outer(g, carry):
        for b in range(2):
            ci = g * 2 + b
            wait_gathers(b)

            @pl.when((ci % CPB == 2) & (ci < (NB - 1) * CPB))
            def _():
                load_block(ci // CPB + 1)

            @pl.when((ci % CPB == CPB - 1) & (ci + 1 < RPW))
            def _():
                wait_block()

            @pl.when(ci + 1 < RPW)
            def _():
                issue_gathers(ci + 1, 1 - b)

            @pl.when(ci >= 2)
            def _():
                drain_scatter(b)

            for j in range(K // 16):
                rows = ii + j * 16
                for h in range(H):
                    ch = jnp.full((16,), h, jnp.int32)
                    e = (plsc.load_gather(srb[b], [rows, ch]) +
                         plsc.load_gather(erb[b], [rows, ch]))
                    e = jnp.maximum(e, 0.2 * e)
                    ee = jnp.exp(e - m_v[h])
                    plsc.store_scatter(ob[b], [rows, ch], ee)
                    for f in range(F):
                        c = 4 + h * F + f
                        cc = jnp.full((16,), c, jnp.int32)
                        fv = plsc.load_gather(srb[b], [rows, cc])
                        plsc.store_scatter(ob[b], [rows, cc], ee * fv)
            pltpu.async_copy(ob[b], acc.at[dblk.at[ci % (2 * CPB)]], ss[b],
                             add=True)
        return carry

    lax.fori_loop(0, RPW // 2, outer, 0)
    drain_scatter(0)
    drain_scatter(1)

    plsc.subcore_barrier()
    pltpu.sync_copy(acc.at[pl.ds(sid * NPT, NPT)],
                    out_hbm.at[pl.ds(wid * NPT, NPT)])


def _edge(src2d, dst2d, src_tab, er_tab, m, zeros):
    mesh = plsc.VectorSubcoreMesh(core_axis_name="c", subcore_axis_name="s")
    f = functools.partial(
        pl.kernel,
        out_type=jax.ShapeDtypeStruct((NC * N, ACCW), jnp.float32),
        mesh=mesh,
        scratch_types=[
            pltpu.VMEM((2 * CPB, K), jnp.int32),
            pltpu.VMEM((2 * CPB, K), jnp.int32),
            pltpu.VMEM((K, SRCW), jnp.float32),
            pltpu.VMEM((K, ERW), jnp.float32),
            pltpu.VMEM((K, SRCW), jnp.float32),
            pltpu.VMEM((K, ERW), jnp.float32),
            pltpu.VMEM((K, ACCW), jnp.float32),
            pltpu.VMEM((K, ACCW), jnp.float32),
            pltpu.VMEM((H, 16), jnp.float32),
            pltpu.VMEM_SHARED((N, ACCW), jnp.float32),
            pltpu.SemaphoreType.DMA,
            pltpu.SemaphoreType.DMA,
            pltpu.SemaphoreType.DMA,
            pltpu.SemaphoreType.DMA,
            pltpu.SemaphoreType.DMA,
        ],
        compiler_params=pltpu.CompilerParams(needs_layout_passes=False,
                                             use_tc_tiling_on_sc=False),
    )(_edge_body)
    return f(src2d, dst2d, src_tab, er_tab, m, zeros)


GB = 64
GN = GB * C


def _post_body(p0, p1, gat_bias, proj_W, proj_b, gw1, gw2, w_ref, out_ref):
    accb = p0[...] + p1[...]
    den = accb[:, 0:4]
    msg = accb[:, 4:4 + HF]
    hrow = lax.broadcasted_iota(jnp.int32, (H, HF), 0)
    hcol = lax.broadcasted_iota(jnp.int32, (H, HF), 1) // F
    R = jnp.where(hrow == hcol, 1.0, 0.0)
    den16 = den @ R
    rst = jnp.where(den16 != 0.0, msg / den16, 0.0) + gat_bias[...]
    encode = rst.reshape(GB, C, HF)
    graph1 = lax.dot_general(encode, encode,
                             (((2,), (2,)), ((0,), (0,))))
    h = lax.dot_general(encode, proj_W[...],
                        (((2,), (0,)), ((), ()))) + proj_b[...][None]
    u = lax.dot_general(h, gw1[...], (((2,), (0,)), ((), ())))
    v = lax.dot_general(h, gw2[...], (((2,), (0,)), ((), ())))
    graph2 = u + v[:, :, 0][:, None, :]
    w = w_ref[...][None]
    out_ref[...] = w * graph1 + (1.0 - w) * graph2


def _post(partial, gat_bias, proj_W, proj_b, gw1, gw2, w_param):
    grid = B // GB
    small = lambda shape: pl.BlockSpec(shape, lambda i: (0,) * len(shape))
    return pl.pallas_call(
        _post_body,
        grid=(grid,),
        in_specs=[
            pl.BlockSpec((GN, ACCW), lambda i: (i, 0)),
            pl.BlockSpec((GN, ACCW), lambda i: (i + grid, 0)),
            small((1, HF)),
            small((EMB, GH)),
            small((1, GH)),
            small((GH, 1)),
            small((GH, 1)),
            small((C, C)),
        ],
        out_specs=pl.BlockSpec((GB, C, C), lambda i: (i, 0, 0)),
        out_shape=jax.ShapeDtypeStruct((B, C, C), jnp.float32),
    )(partial, partial, gat_bias, proj_W, proj_b, gw1, gw2, w_param)


def kernel(feature, edge_index, emb_W, emb_b, se_W1, se_b1, se_W2, se_b2,
           fc_W, attn_l, attn_r, gat_bias, proj_W, proj_b, graph_W, w_param):
    src = edge_index[0].reshape(E // K, K)
    dst = edge_index[1].reshape(E // K, K)
    eye = jnp.eye(H, dtype=jnp.float32)
    AL = (eye[:, None, :] * attn_l[:, :, None]).reshape(HF, H)
    AR = (eye[:, None, :] * attn_r[:, :, None]).reshape(HF, H)

    src_tab, er_tab, m = _prep(feature, emb_W, emb_b.reshape(1, EMB),
                               se_W1, se_b1.reshape(1, 4), se_W2,
                               se_b2.reshape(1, EMB), fc_W, AL, AR)
    zeros = jnp.zeros((NPT, ACCW), jnp.float32)
    partial = _edge(src, dst, src_tab, er_tab, m, zeros)
    return _post(partial, gat_bias.reshape(1, HF), proj_W,
                 proj_b.reshape(1, GH), graph_W[:GH], graph_W[GH:], w_param)


# speedup vs baseline: 227.2874x; 1.4374x over previous
import functools

import jax
import jax.numpy as jnp
from jax import lax
from jax.experimental import pallas as pl
from jax.experimental.pallas import tpu as pltpu
from jax.experimental.pallas import tpu_sc as plsc

B, C, L = 512, 64, 96
EMB = 16
H, F = 4, 4
HF = H * F
GH = 4
N = B * C
E = 1048576

NC, NS = 2, 16
NW = NC * NS
EPW = E // NW
K = 128
SRCW = 24
ERW = 4
ACCW = 24
NPT = N // NS


PB = 128
PN = PB * C


def _prep_body(feature, emb_W, emb_b, se_W1, se_b1, se_W2, se_b2, fc_W,
               AL, AR, src_tab, er_tab, m_ref):
    i = pl.program_id(0)
    x = feature[...].reshape(PN, L) @ emb_W[...] + emb_b[...]
    x3 = x.reshape(PB, C, EMB)
    s = jnp.mean(x3, axis=1)
    a = jax.nn.relu(s @ se_W1[...] + se_b1[...])
    g = jax.nn.sigmoid(a @ se_W2[...] + se_b2[...])
    embed = (x3 * g[:, None, :]).reshape(PN, EMB)
    feat = embed @ fc_W[...]
    el = feat @ AL[...]
    er = feat @ AR[...]
    mel = jnp.broadcast_to(jnp.max(el, axis=0)[:, None], (H, 16))
    mer = jnp.broadcast_to(jnp.max(er, axis=0)[:, None], (H, 16))
    mb = jnp.concatenate([mel, mer], axis=0)

    @pl.when(i == 0)
    def _():
        m_ref[...] = mb

    @pl.when(i > 0)
    def _():
        m_ref[...] = jnp.maximum(m_ref[...], mb)

    @pl.when(i == B // PB - 1)
    def _():
        m4 = m_ref[0:4, :] + m_ref[4:8, :]
        m_ref[0:4, :] = jnp.maximum(m4, 0.2 * m4)

    pad4 = jnp.zeros((PN, 4), jnp.float32)
    src_tab[...] = jnp.concatenate([el, feat, pad4], axis=1)
    er_tab[...] = er


def _prep(feature, emb_W, emb_b, se_W1, se_b1, se_W2, se_b2, fc_W, AL, AR):
    grid = B // PB
    small = lambda shape: pl.BlockSpec(shape, lambda i: (0,) * len(shape))
    return pl.pallas_call(
        _prep_body,
        grid=(grid,),
        in_specs=[
            pl.BlockSpec((PB, C, L), lambda i: (i, 0, 0)),
            small((L, EMB)),
            small((1, EMB)),
            small((EMB, 4)),
            small((1, 4)),
            small((4, EMB)),
            small((1, EMB)),
            small((EMB, HF)),
            small((HF, H)),
            small((HF, H)),
        ],
        out_specs=[
            pl.BlockSpec((PN, SRCW), lambda i: (i, 0)),
            pl.BlockSpec((PN, ERW), lambda i: (i, 0)),
            pl.BlockSpec((8, 16), lambda i: (0, 0)),
        ],
        out_shape=[
            jax.ShapeDtypeStruct((N, SRCW), jnp.float32),
            jax.ShapeDtypeStruct((N, ERW), jnp.float32),
            jax.ShapeDtypeStruct((8, 16), jnp.float32),
        ],
    )(feature, emb_W, emb_b, se_W1, se_b1, se_W2, se_b2, fc_W, AL, AR)


CPB = 16
RPW = EPW // K
NB = RPW // CPB


def _edge_body(src_hbm, dst_hbm, src_tab, er_tab, m_hbm, zeros_hbm, out_hbm,
               sblk, dblk, sr0, erb0, sr1, erb1,
               ob0, ob1, m_v, acc, si, sg0, sg1, ss0, ss1):
    cid = lax.axis_index("c")
    sid = lax.axis_index("s")
    wid = cid * NS + sid

    srb = (sr0, sr1)
    erb = (erb0, erb1)
    ob = (ob0, ob1)
    sg = (sg0, sg1)
    ss = (ss0, ss1)

    pltpu.sync_copy(m_hbm.at[pl.ds(0, H)], m_v)
    pltpu.sync_copy(zeros_hbm, acc.at[pl.ds(sid * NPT, NPT)])
    plsc.subcore_barrier()

    ii = lax.iota(jnp.int32, 16)
    zero16 = jnp.zeros((16,), jnp.float32)
    for buf in ob:
        for j in range(K // 16):
            rows = ii + j * 16
            for c in range(4 + HF, ACCW):
                plsc.store_scatter(buf, [rows, jnp.full((16,), c, jnp.int32)],
                                   zero16)

    rbase = wid * RPW

    def load_block(t):
        half = (t % 2) * CPB
        pltpu.async_copy(src_hbm.at[pl.ds(rbase + t * CPB, CPB)],
                         sblk.at[pl.ds(half, CPB)], si)
        pltpu.async_copy(dst_hbm.at[pl.ds(rbase + t * CPB, CPB)],
                         dblk.at[pl.ds(half, CPB)], si)

    def wait_block():
        pltpu.make_async_copy(src_hbm.at[pl.ds(0, CPB)],
                              sblk.at[pl.ds(0, CPB)], si).wait()
        pltpu.make_async_copy(dst_hbm.at[pl.ds(0, CPB)],
                              dblk.at[pl.ds(0, CPB)], si).wait()

    def issue_gathers(ci, b):
        r = ci % (2 * CPB)
        pltpu.async_copy(src_tab.at[sblk.at[r]], srb[b], sg[b])
        pltpu.async_copy(er_tab.at[dblk.at[r]], erb[b], sg[b])

    def wait_gathers(b):
        pltpu.make_async_copy(src_tab.at[sblk.at[0]], srb[b], sg[b]).wait()
        pltpu.make_async_copy(er_tab.at[dblk.at[0]], erb[b], sg[b]).wait()

    def drain_scatter(b):
        pltpu.make_async_copy(ob[b], acc.at[dblk.at[0]], ss[b]).wait()

    load_block(0)
    wait_block()
    issue_gathers(0, 0)

    def outer(g, carry):
        for b in range(2):
            ci = g * 2 + b
            wait_gathers(b)

            @pl.when((ci % CPB == 2) & (ci < (NB - 1) * CPB))
            def _():
                load_block(ci // CPB + 1)

            @pl.when((ci % CPB == CPB - 1) & (ci + 1 < RPW))
            def _():
                wait_block()

            @pl.when(ci + 1 < RPW)
            def _():
                issue_gathers(ci + 1, 1 - b)

            @pl.when(ci >= 2)
            def _():
                drain_scatter(b)

            for j in range(K // 16):
                rows = ii + j * 16
                for h in range(H):
                    ch = jnp.full((16,), h, jnp.int32)
                    e = (plsc.load_gather(srb[b], [rows, ch]) +
                         plsc.load_gather(erb[b], [rows, ch]))
                    e = jnp.maximum(e, 0.2 * e)
                    ee = jnp.exp(e - m_v[h])
                    plsc.store_scatter(ob[b], [rows, ch], ee)
                    for f in range(F):
                        c = 4 + h * F + f
                        cc = jnp.full((16,), c, jnp.int32)
                        fv = plsc.load_gather(srb[b], [rows, cc])
                        plsc.store_scatter(ob[b], [rows, cc], ee * fv)
            pltpu.async_copy(ob[b], acc.at[dblk.at[ci % (2 * CPB)]], ss[b],
                             add=True)
        return carry

    lax.fori_loop(0, RPW // 2, outer, 0)
    drain_scatter(0)
    drain_scatter(1)

    plsc.subcore_barrier()
    pltpu.sync_copy(acc.at[pl.ds(sid * NPT, NPT)],
                    out_hbm.at[pl.ds(wid * NPT, NPT)])


def _edge(src2d, dst2d, src_tab, er_tab, m, zeros):
    mesh = plsc.VectorSubcoreMesh(core_axis_name="c", subcore_axis_name="s")
    f = functools.partial(
        pl.kernel,
        out_type=jax.ShapeDtypeStruct((NC * N, ACCW), jnp.float32),
        mesh=mesh,
        scratch_types=[
            pltpu.VMEM((2 * CPB, K), jnp.int32),
            pltpu.VMEM((2 * CPB, K), jnp.int32),
            pltpu.VMEM((K, SRCW), jnp.float32),
            pltpu.VMEM((K, ERW), jnp.float32),
            pltpu.VMEM((K, SRCW), jnp.float32),
            pltpu.VMEM((K, ERW), jnp.float32),
            pltpu.VMEM((K, ACCW), jnp.float32),
            pltpu.VMEM((K, ACCW), jnp.float32),
            pltpu.VMEM((H, 16), jnp.float32),
            pltpu.VMEM_SHARED((N, ACCW), jnp.float32),
            pltpu.SemaphoreType.DMA,
            pltpu.SemaphoreType.DMA,
            pltpu.SemaphoreType.DMA,
            pltpu.SemaphoreType.DMA,
            pltpu.SemaphoreType.DMA,
        ],
        compiler_params=pltpu.CompilerParams(needs_layout_passes=False,
                                             use_tc_tiling_on_sc=False),
    )(_edge_body)
    return f(src2d, dst2d, src_tab, er_tab, m, zeros)


GB = 64
GN = GB * C


def _post_body(p0, p1, gat_bias, proj_W, proj_b, gw1, gw2, w_ref, out_ref):
    accb = p0[...] + p1[...]
    den = accb[:, 0:4]
    msg = accb[:, 4:4 + HF]
    hrow = lax.broadcasted_iota(jnp.int32, (H, HF), 0)
    hcol = lax.broadcasted_iota(jnp.int32, (H, HF), 1) // F
    R = jnp.where(hrow == hcol, 1.0, 0.0)
    den16 = den @ R
    rst = jnp.where(den16 != 0.0, msg / den16, 0.0) + gat_bias[...]
    encode = rst.reshape(GB, C, HF)
    graph1 = lax.dot_general(encode, encode,
                             (((2,), (2,)), ((0,), (0,))))
    h = lax.dot_general(encode, proj_W[...],
                        (((2,), (0,)), ((), ()))) + proj_b[...][None]
    u = lax.dot_general(h, gw1[...], (((2,), (0,)), ((), ())))
    v = lax.dot_general(h, gw2[...], (((2,), (0,)), ((), ())))
    graph2 = u + v[:, :, 0][:, None, :]
    w = w_ref[...][None]
    out_ref[...] = w * graph1 + (1.0 - w) * graph2


def _post(partial, gat_bias, proj_W, proj_b, gw1, gw2, w_param):
    grid = B // GB
    small = lambda shape: pl.BlockSpec(shape, lambda i: (0,) * len(shape))
    return pl.pallas_call(
        _post_body,
        grid=(grid,),
        in_specs=[
            pl.BlockSpec((GN, ACCW), lambda i: (i, 0)),
            pl.BlockSpec((GN, ACCW), lambda i: (i + grid, 0)),
            small((1, HF)),
            small((EMB, GH)),
            small((1, GH)),
            small((GH, 1)),
            small((GH, 1)),
            small((C, C)),
        ],
        out_specs=pl.BlockSpec((GB, C, C), lambda i: (i, 0, 0)),
        out_shape=jax.ShapeDtypeStruct((B, C, C), jnp.float32),
    )(partial, partial, gat_bias, proj_W, proj_b, gw1, gw2, w_param)


def kernel(feature, edge_index, emb_W, emb_b, se_W1, se_b1, se_W2, se_b2,
           fc_W, attn_l, attn_r, gat_bias, proj_W, proj_b, graph_W, w_param):
    src = edge_index[0].reshape(E // K, K)
    dst = edge_index[1].reshape(E // K, K)
    eye = jnp.eye(H, dtype=jnp.float32)
    AL = (eye[:, None, :] * attn_l[:, :, None]).reshape(HF, H)
    AR = (eye[:, None, :] * attn_r[:, :, None]).reshape(HF, H)

    src_tab, er_tab, m = _prep(feature, emb_W, emb_b.reshape(1, EMB),
                               se_W1, se_b1.reshape(1, 4), se_W2,
                               se_b2.reshape(1, EMB), fc_W, AL, AR)
    zeros = jnp.zeros((NPT, ACCW), jnp.float32)
    partial = _edge(src, dst, src_tab, er_tab, m, zeros)
    return _post(partial, gat_bias.reshape(1, HF), proj_W,
                 proj_b.reshape(1, GH), graph_W[:GH], graph_W[GH:], w_param)


# speedup vs baseline: 230.6117x; 1.0146x over previous
import functools

import jax
import jax.numpy as jnp
from jax import lax
from jax.experimental import pallas as pl
from jax.experimental.pallas import tpu as pltpu
from jax.experimental.pallas import tpu_sc as plsc

B, C, L = 512, 64, 96
EMB = 16
H, F = 4, 4
HF = H * F
GH = 4
N = B * C
E = 1048576

NC, NS = 2, 16
NW = NC * NS
EPW = E // NW
K = 128
SRCW = 24
ERW = 8
ACCW = 24
NPT = N // NS


PB = 128
PN = PB * C


def _prep_body(feature, emb_W, emb_b, se_W1, se_b1, se_W2, se_b2, fc_W,
               AL, AR, src_tab, er_tab, m_ref):
    i = pl.program_id(0)
    x = feature[...].reshape(PN, L) @ emb_W[...] + emb_b[...]
    x3 = x.reshape(PB, C, EMB)
    s = jnp.mean(x3, axis=1)
    a = jax.nn.relu(s @ se_W1[...] + se_b1[...])
    g = jax.nn.sigmoid(a @ se_W2[...] + se_b2[...])
    embed = (x3 * g[:, None, :]).reshape(PN, EMB)
    feat = embed @ fc_W[...]
    el = feat @ AL[...]
    er = feat @ AR[...]
    mel = jnp.broadcast_to(jnp.max(el, axis=0)[:, None], (H, 16))
    mer = jnp.broadcast_to(jnp.max(er, axis=0)[:, None], (H, 16))
    mb = jnp.concatenate([mel, mer], axis=0)

    @pl.when(i == 0)
    def _():
        m_ref[...] = mb

    @pl.when(i > 0)
    def _():
        m_ref[...] = jnp.maximum(m_ref[...], mb)

    @pl.when(i == B // PB - 1)
    def _():
        m4 = m_ref[0:4, :] + m_ref[4:8, :]
        m_ref[0:4, :] = jnp.maximum(m4, 0.2 * m4)

    pad4 = jnp.zeros((PN, 4), jnp.float32)
    src_tab[...] = jnp.concatenate([el, feat, pad4], axis=1)
    er_tab[...] = jnp.concatenate([er, pad4], axis=1)


def _prep(feature, emb_W, emb_b, se_W1, se_b1, se_W2, se_b2, fc_W, AL, AR):
    grid = B // PB
    small = lambda shape: pl.BlockSpec(shape, lambda i: (0,) * len(shape))
    return pl.pallas_call(
        _prep_body,
        grid=(grid,),
        in_specs=[
            pl.BlockSpec((PB, C, L), lambda i: (i, 0, 0)),
            small((L, EMB)),
            small((1, EMB)),
            small((EMB, 4)),
            small((1, 4)),
            small((4, EMB)),
            small((1, EMB)),
            small((EMB, HF)),
            small((HF, H)),
            small((HF, H)),
        ],
        out_specs=[
            pl.BlockSpec((PN, SRCW), lambda i: (i, 0)),
            pl.BlockSpec((PN, ERW), lambda i: (i, 0)),
            pl.BlockSpec((8, 16), lambda i: (0, 0)),
        ],
        out_shape=[
            jax.ShapeDtypeStruct((N, SRCW), jnp.float32),
            jax.ShapeDtypeStruct((N, ERW), jnp.float32),
            jax.ShapeDtypeStruct((8, 16), jnp.float32),
        ],
    )(feature, emb_W, emb_b, se_W1, se_b1, se_W2, se_b2, fc_W, AL, AR)


CPB = 16
RPW = EPW // K
NB = RPW // CPB


def _edge_body(src_hbm, dst_hbm, src_tab, er_tab, m_hbm, zeros_hbm, out_hbm,
               sblk, dblk, sr0, erb0, sr1, erb1,
               ob0, ob1, m_v, acc, si, sg0, sg1, ss0, ss1):
    cid = lax.axis_index("c")
    sid = lax.axis_index("s")
    wid = cid * NS + sid

    srb = (sr0, sr1)
    erb = (erb0, erb1)
    ob = (ob0, ob1)
    sg = (sg0, sg1)
    ss = (ss0, ss1)

    pltpu.sync_copy(m_hbm.at[pl.ds(0, H)], m_v)
    pltpu.sync_copy(zeros_hbm, acc.at[pl.ds(sid * NPT, NPT)])
    plsc.subcore_barrier()

    ii = lax.iota(jnp.int32, 16)
    zero16 = jnp.zeros((16,), jnp.float32)
    for buf in ob:
        for j in range(K // 16):
            rows = ii + j * 16
            for c in range(4 + HF, ACCW):
                plsc.store_scatter(buf, [rows, jnp.full((16,), c, jnp.int32)],
                                   zero16)

    rbase = wid * RPW

    def load_block(t):
        half = (t % 2) * CPB
        pltpu.async_copy(src_hbm.at[pl.ds(rbase + t * CPB, CPB)],
                         sblk.at[pl.ds(half, CPB)], si)
        pltpu.async_copy(dst_hbm.at[pl.ds(rbase + t * CPB, CPB)],
                         dblk.at[pl.ds(half, CPB)], si)

    def wait_block():
        pltpu.make_async_copy(src_hbm.at[pl.ds(0, CPB)],
                              sblk.at[pl.ds(0, CPB)], si).wait()
        pltpu.make_async_copy(dst_hbm.at[pl.ds(0, CPB)],
                              dblk.at[pl.ds(0, CPB)], si).wait()

    def issue_gathers(ci, b):
        r = ci % (2 * CPB)
        pltpu.async_copy(src_tab.at[sblk.at[r]], srb[b], sg[b])
        pltpu.async_copy(er_tab.at[dblk.at[r]], erb[b], sg[b])

    def wait_gathers(b):
        pltpu.make_async_copy(src_tab.at[sblk.at[0]], srb[b], sg[b]).wait()
        pltpu.make_async_copy(er_tab.at[dblk.at[0]], erb[b], sg[b]).wait()

    def drain_scatter(b):
        pltpu.make_async_copy(ob[b], acc.at[dblk.at[0]], ss[b]).wait()

    load_block(0)
    wait_block()
    issue_gathers(0, 0)

    def outer(g, carry):
        for b in range(2):
            ci = g * 2 + b
            wait_gathers(b)

            @pl.when((ci % CPB == 2) & (ci < (NB - 1) * CPB))
            def _():
                load_block(ci // CPB + 1)

            @pl.when((ci % CPB == CPB - 1) & (ci + 1 < RPW))
            def _():
                wait_block()

            @pl.when(ci + 1 < RPW)
            def _():
                issue_gathers(ci + 1, 1 - b)

            @pl.when(ci >= 2)
            def _():
                drain_scatter(b)

            for j in range(K // 16):
                rows = ii + j * 16
                for h in range(H):
                    ch = jnp.full((16,), h, jnp.int32)
                    e = (plsc.load_gather(srb[b], [rows, ch]) +
                         plsc.load_gather(erb[b], [rows, ch]))
                    e = jnp.maximum(e, 0.2 * e)
                    ee = jnp.exp(e - m_v[h])
                    plsc.store_scatter(ob[b], [rows, ch], ee)
                    for f in range(F):
                        c = 4 + h * F + f
                        cc = jnp.full((16,), c, jnp.int32)
                        fv = plsc.load_gather(srb[b], [rows, cc])
                        plsc.store_scatter(ob[b], [rows, cc], ee * fv)
            pltpu.async_copy(ob[b], acc.at[dblk.at[ci % (2 * CPB)]], ss[b],
                             add=True)
        return carry

    lax.fori_loop(0, RPW // 2, outer, 0)
    drain_scatter(0)
    drain_scatter(1)

    plsc.subcore_barrier()
    pltpu.sync_copy(acc.at[pl.ds(sid * NPT, NPT)],
                    out_hbm.at[pl.ds(wid * NPT, NPT)])


def _edge(src2d, dst2d, src_tab, er_tab, m, zeros):
    mesh = plsc.VectorSubcoreMesh(core_axis_name="c", subcore_axis_name="s")
    f = functools.partial(
        pl.kernel,
        out_type=jax.ShapeDtypeStruct((NC * N, ACCW), jnp.float32),
        mesh=mesh,
        scratch_types=[
            pltpu.VMEM((2 * CPB, K), jnp.int32),
            pltpu.VMEM((2 * CPB, K), jnp.int32),
            pltpu.VMEM((K, SRCW), jnp.float32),
            pltpu.VMEM((K, ERW), jnp.float32),
            pltpu.VMEM((K, SRCW), jnp.float32),
            pltpu.VMEM((K, ERW), jnp.float32),
            pltpu.VMEM((K, ACCW), jnp.float32),
            pltpu.VMEM((K, ACCW), jnp.float32),
            pltpu.VMEM((H, 16), jnp.float32),
            pltpu.VMEM_SHARED((N, ACCW), jnp.float32),
            pltpu.SemaphoreType.DMA,
            pltpu.SemaphoreType.DMA,
            pltpu.SemaphoreType.DMA,
            pltpu.SemaphoreType.DMA,
            pltpu.SemaphoreType.DMA,
        ],
        compiler_params=pltpu.CompilerParams(needs_layout_passes=False,
                                             use_tc_tiling_on_sc=False),
    )(_edge_body)
    return f(src2d, dst2d, src_tab, er_tab, m, zeros)


GB = 64
GN = GB * C


def _post_body(p0, p1, gat_bias, proj_W, proj_b, gw1, gw2, w_ref, out_ref):
    accb = p0[...] + p1[...]
    den = accb[:, 0:4]
    msg = accb[:, 4:4 + HF]
    hrow = lax.broadcasted_iota(jnp.int32, (H, HF), 0)
    hcol = lax.broadcasted_iota(jnp.int32, (H, HF), 1) // F
    R = jnp.where(hrow == hcol, 1.0, 0.0)
    den16 = den @ R
    rst = jnp.where(den16 != 0.0, msg / den16, 0.0) + gat_bias[...]
    encode = rst.reshape(GB, C, HF)
    graph1 = lax.dot_general(encode, encode,
                             (((2,), (2,)), ((0,), (0,))))
    h = lax.dot_general(encode, proj_W[...],
                        (((2,), (0,)), ((), ()))) + proj_b[...][None]
    u = lax.dot_general(h, gw1[...], (((2,), (0,)), ((), ())))
    v = lax.dot_general(h, gw2[...], (((2,), (0,)), ((), ())))
    graph2 = u + v[:, :, 0][:, None, :]
    w = w_ref[...][None]
    out_ref[...] = w * graph1 + (1.0 - w) * graph2


def _post(partial, gat_bias, proj_W, proj_b, gw1, gw2, w_param):
    grid = B // GB
    small = lambda shape: pl.BlockSpec(shape, lambda i: (0,) * len(shape))
    return pl.pallas_call(
        _post_body,
        grid=(grid,),
        in_specs=[
            pl.BlockSpec((GN, ACCW), lambda i: (i, 0)),
            pl.BlockSpec((GN, ACCW), lambda i: (i + grid, 0)),
            small((1, HF)),
            small((EMB, GH)),
            small((1, GH)),
            small((GH, 1)),
            small((GH, 1)),
            small((C, C)),
        ],
        out_specs=pl.BlockSpec((GB, C, C), lambda i: (i, 0, 0)),
        out_shape=jax.ShapeDtypeStruct((B, C, C), jnp.float32),
    )(partial, partial, gat_bias, proj_W, proj_b, gw1, gw2, w_param)


def kernel(feature, edge_index, emb_W, emb_b, se_W1, se_b1, se_W2, se_b2,
           fc_W, attn_l, attn_r, gat_bias, proj_W, proj_b, graph_W, w_param):
    src = edge_index[0].reshape(E // K, K)
    dst = edge_index[1].reshape(E // K, K)
    eye = jnp.eye(H, dtype=jnp.float32)
    AL = (eye[:, None, :] * attn_l[:, :, None]).reshape(HF, H)
    AR = (eye[:, None, :] * attn_r[:, :, None]).reshape(HF, H)

    src_tab, er_tab, m = _prep(feature, emb_W, emb_b.reshape(1, EMB),
                               se_W1, se_b1.reshape(1, 4), se_W2,
                               se_b2.reshape(1, EMB), fc_W, AL, AR)
    zeros = jnp.zeros((NPT, ACCW), jnp.float32)
    partial = _edge(src, dst, src_tab, er_tab, m, zeros)
    return _post(partial, gat_bias.reshape(1, HF), proj_W,
                 proj_b.reshape(1, GH), graph_W[:GH], graph_W[GH:], w_param)


# speedup vs baseline: 345.9335x; 1.5001x over previous
import functools

import jax
import jax.numpy as jnp
from jax import lax
from jax.experimental import pallas as pl
from jax.experimental.pallas import tpu as pltpu
from jax.experimental.pallas import tpu_sc as plsc

B, C, L = 512, 64, 96
EMB = 16
H, F = 4, 4
HF = H * F
GH = 4
N = B * C
E = 1048576

NC, NS = 2, 16
NW = NC * NS
EPW = E // NW
K = 128
SRCW = 24
ERW = 8
ACCW = 24
NPT = N // NS


PB = 128
PN = PB * C


def _prep_body(feature, emb_W, emb_b, se_W1, se_b1, se_W2, se_b2, fc_W,
               AL, AR, src_tab, er_tab, m_ref):
    i = pl.program_id(0)
    x = feature[...].reshape(PN, L) @ emb_W[...] + emb_b[...]
    x3 = x.reshape(PB, C, EMB)
    s = jnp.mean(x3, axis=1)
    a = jax.nn.relu(s @ se_W1[...] + se_b1[...])
    g = jax.nn.sigmoid(a @ se_W2[...] + se_b2[...])
    embed = (x3 * g[:, None, :]).reshape(PN, EMB)
    feat = embed @ fc_W[...]
    el = feat @ AL[...]
    er = feat @ AR[...]
    mel = jnp.broadcast_to(jnp.max(el, axis=0)[:, None], (H, 16))
    mer = jnp.broadcast_to(jnp.max(er, axis=0)[:, None], (H, 16))
    mb = jnp.concatenate([mel, mer], axis=0)

    @pl.when(i == 0)
    def _():
        m_ref[...] = mb

    @pl.when(i > 0)
    def _():
        m_ref[...] = jnp.maximum(m_ref[...], mb)

    @pl.when(i == B // PB - 1)
    def _():
        m4 = m_ref[0:4, :] + m_ref[4:8, :]
        m_ref[0:4, :] = jnp.maximum(m4, 0.2 * m4)

    pad4 = jnp.zeros((PN, 4), jnp.float32)
    src_tab[...] = jnp.concatenate([el, feat, pad4], axis=1)
    er_tab[...] = jnp.concatenate([er, pad4], axis=1)


def _prep(feature, emb_W, emb_b, se_W1, se_b1, se_W2, se_b2, fc_W, AL, AR):
    grid = B // PB
    small = lambda shape: pl.BlockSpec(shape, lambda i: (0,) * len(shape))
    return pl.pallas_call(
        _prep_body,
        grid=(grid,),
        in_specs=[
            pl.BlockSpec((PB, C, L), lambda i: (i, 0, 0)),
            small((L, EMB)),
            small((1, EMB)),
            small((EMB, 4)),
            small((1, 4)),
            small((4, EMB)),
            small((1, EMB)),
            small((EMB, HF)),
            small((HF, H)),
            small((HF, H)),
        ],
        out_specs=[
            pl.BlockSpec((PN, SRCW), lambda i: (i, 0)),
            pl.BlockSpec((PN, ERW), lambda i: (i, 0)),
            pl.BlockSpec((8, 16), lambda i: (0, 0)),
        ],
        out_shape=[
            jax.ShapeDtypeStruct((N, SRCW), jnp.float32),
            jax.ShapeDtypeStruct((N, ERW), jnp.float32),
            jax.ShapeDtypeStruct((8, 16), jnp.float32),
        ],
    )(feature, emb_W, emb_b, se_W1, se_b1, se_W2, se_b2, fc_W, AL, AR)


CPB = 16
RPW = EPW // K
NB = RPW // CPB


def _edge_body(src_hbm, dst_hbm, src_tab, er_tab, m_hbm, zeros_hbm, out_hbm,
               sblk, dblk, sr0, erb0, sr1, erb1,
               ob0, ob1, m_v, acc, si, sg0, sg1, ss0, ss1):
    cid = lax.axis_index("c")
    sid = lax.axis_index("s")
    wid = cid * NS + sid

    srb = (sr0, sr1)
    erb = (erb0, erb1)
    ob = (ob0, ob1)
    sg = (sg0, sg1)
    ss = (ss0, ss1)

    pltpu.sync_copy(m_hbm.at[pl.ds(0, H)], m_v)
    pltpu.sync_copy(zeros_hbm, acc.at[pl.ds(sid * NPT, NPT)])
    plsc.subcore_barrier()

    ii = lax.iota(jnp.int32, 16)
    zero16 = jnp.zeros((16,), jnp.float32)
    for buf in ob:
        for j in range(K // 16):
            rows = ii + j * 16
            for c in range(4 + HF, ACCW):
                plsc.store_scatter(buf, [rows, jnp.full((16,), c, jnp.int32)],
                                   zero16)

    rbase = wid * RPW

    def load_block(t):
        half = (t % 2) * CPB
        pltpu.async_copy(src_hbm.at[pl.ds(rbase + t * CPB, CPB)],
                         sblk.at[pl.ds(half, CPB)], si)
        pltpu.async_copy(dst_hbm.at[pl.ds(rbase + t * CPB, CPB)],
                         dblk.at[pl.ds(half, CPB)], si)

    def wait_block():
        pltpu.make_async_copy(src_hbm.at[pl.ds(0, CPB)],
                              sblk.at[pl.ds(0, CPB)], si).wait()
        pltpu.make_async_copy(dst_hbm.at[pl.ds(0, CPB)],
                              dblk.at[pl.ds(0, CPB)], si).wait()

    def issue_gathers(ci, b):
        r = ci % (2 * CPB)
        pltpu.async_copy(src_tab.at[sblk.at[r]], srb[b], sg[b])
        pltpu.async_copy(er_tab.at[dblk.at[r]], erb[b], sg[b])

    def wait_gathers(b):
        pltpu.make_async_copy(src_tab.at[sblk.at[0]], srb[b], sg[b]).wait()
        pltpu.make_async_copy(er_tab.at[dblk.at[0]], erb[b], sg[b]).wait()

    def drain_scatter(b):
        pltpu.make_async_copy(ob[b], acc.at[dblk.at[0]], ss[b]).wait()

    load_block(0)
    wait_block()
    issue_gathers(0, 0)

    def outer(g, carry):
        for b in range(2):
            ci = g * 2 + b
            wait_gathers(b)

            @pl.when((ci % CPB == 2) & (ci < (NB - 1) * CPB))
            def _():
                load_block(ci // CPB + 1)

            @pl.when((ci % CPB == CPB - 1) & (ci + 1 < RPW))
            def _():
                wait_block()

            @pl.when(ci + 1 < RPW)
            def _():
                issue_gathers(ci + 1, 1 - b)

            @pl.when(ci >= 2)
            def _():
                drain_scatter(b)

            for j in range(K // 16):
                rows = ii + j * 16
                chs = [jnp.full((16,), h, jnp.int32) for h in range(H)]
                els = [plsc.load_gather(srb[b], [rows, chs[h]])
                       for h in range(H)]
                ers = [plsc.load_gather(erb[b], [rows, chs[h]])
                      for h in range(H)]
                fvs = [plsc.load_gather(
                           srb[b], [rows, jnp.full((16,), 4 + c, jnp.int32)])
                       for c in range(HF)]
                es = [els[h] + ers[h] for h in range(H)]
                es = [jnp.maximum(e, 0.2 * e) for e in es]
                ees = [jnp.exp(es[h] - m_v[h]) for h in range(H)]
                for h in range(H):
                    plsc.store_scatter(ob[b], [rows, chs[h]], ees[h])
                outs = [ees[c // F] * fvs[c] for c in range(HF)]
                for c in range(HF):
                    plsc.store_scatter(
                        ob[b], [rows, jnp.full((16,), 4 + c, jnp.int32)],
                        outs[c])
            pltpu.async_copy(ob[b], acc.at[dblk.at[ci % (2 * CPB)]], ss[b],
                             add=True)
        return carry

    lax.fori_loop(0, RPW // 2, outer, 0)
    drain_scatter(0)
    drain_scatter(1)

    plsc.subcore_barrier()
    pltpu.sync_copy(acc.at[pl.ds(sid * NPT, NPT)],
                    out_hbm.at[pl.ds(wid * NPT, NPT)])


def _edge(src2d, dst2d, src_tab, er_tab, m, zeros):
    mesh = plsc.VectorSubcoreMesh(core_axis_name="c", subcore_axis_name="s")
    f = functools.partial(
        pl.kernel,
        out_type=jax.ShapeDtypeStruct((NC * N, ACCW), jnp.float32),
        mesh=mesh,
        scratch_types=[
            pltpu.VMEM((2 * CPB, K), jnp.int32),
            pltpu.VMEM((2 * CPB, K), jnp.int32),
            pltpu.VMEM((K, SRCW), jnp.float32),
            pltpu.VMEM((K, ERW), jnp.float32),
            pltpu.VMEM((K, SRCW), jnp.float32),
            pltpu.VMEM((K, ERW), jnp.float32),
            pltpu.VMEM((K, ACCW), jnp.float32),
            pltpu.VMEM((K, ACCW), jnp.float32),
            pltpu.VMEM((H, 16), jnp.float32),
            pltpu.VMEM_SHARED((N, ACCW), jnp.float32),
            pltpu.SemaphoreType.DMA,
            pltpu.SemaphoreType.DMA,
            pltpu.SemaphoreType.DMA,
            pltpu.SemaphoreType.DMA,
            pltpu.SemaphoreType.DMA,
        ],
        compiler_params=pltpu.CompilerParams(needs_layout_passes=False,
                                             use_tc_tiling_on_sc=False),
    )(_edge_body)
    return f(src2d, dst2d, src_tab, er_tab, m, zeros)


GB = 64
GN = GB * C


def _post_body(p0, p1, gat_bias, proj_W, proj_b, gw1, gw2, w_ref, out_ref):
    accb = p0[...] + p1[...]
    den = accb[:, 0:4]
    msg = accb[:, 4:4 + HF]
    hrow = lax.broadcasted_iota(jnp.int32, (H, HF), 0)
    hcol = lax.broadcasted_iota(jnp.int32, (H, HF), 1) // F
    R = jnp.where(hrow == hcol, 1.0, 0.0)
    den16 = den @ R
    rst = jnp.where(den16 != 0.0, msg / den16, 0.0) + gat_bias[...]
    encode = rst.reshape(GB, C, HF)
    graph1 = lax.dot_general(encode, encode,
                             (((2,), (2,)), ((0,), (0,))))
    h = lax.dot_general(encode, proj_W[...],
                        (((2,), (0,)), ((), ()))) + proj_b[...][None]
    u = lax.dot_general(h, gw1[...], (((2,), (0,)), ((), ())))
    v = lax.dot_general(h, gw2[...], (((2,), (0,)), ((), ())))
    graph2 = u + v[:, :, 0][:, None, :]
    w = w_ref[...][None]
    out_ref[...] = w * graph1 + (1.0 - w) * graph2


def _post(partial, gat_bias, proj_W, proj_b, gw1, gw2, w_param):
    grid = B // GB
    small = lambda shape: pl.BlockSpec(shape, lambda i: (0,) * len(shape))
    return pl.pallas_call(
        _post_body,
        grid=(grid,),
        in_specs=[
            pl.BlockSpec((GN, ACCW), lambda i: (i, 0)),
            pl.BlockSpec((GN, ACCW), lambda i: (i + grid, 0)),
            small((1, HF)),
            small((EMB, GH)),
            small((1, GH)),
            small((GH, 1)),
            small((GH, 1)),
            small((C, C)),
        ],
        out_specs=pl.BlockSpec((GB, C, C), lambda i: (i, 0, 0)),
        out_shape=jax.ShapeDtypeStruct((B, C, C), jnp.float32),
    )(partial, partial, gat_bias, proj_W, proj_b, gw1, gw2, w_param)


def kernel(feature, edge_index, emb_W, emb_b, se_W1, se_b1, se_W2, se_b2,
           fc_W, attn_l, attn_r, gat_bias, proj_W, proj_b, graph_W, w_param):
    src = edge_index[0].reshape(E // K, K)
    dst = edge_index[1].reshape(E // K, K)
    eye = jnp.eye(H, dtype=jnp.float32)
    AL = (eye[:, None, :] * attn_l[:, :, None]).reshape(HF, H)
    AR = (eye[:, None, :] * attn_r[:, :, None]).reshape(HF, H)

    src_tab, er_tab, m = _prep(feature, emb_W, emb_b.reshape(1, EMB),
                               se_W1, se_b1.reshape(1, 4), se_W2,
                               se_b2.reshape(1, EMB), fc_W, AL, AR)
    zeros = jnp.zeros((NPT, ACCW), jnp.float32)
    partial = _edge(src, dst, src_tab, er_tab, m, zeros)
    return _post(partial, gat_bias.reshape(1, HF), proj_W,
                 proj_b.reshape(1, GH), graph_W[:GH], graph_W[GH:], w_param)


# speedup vs baseline: 425.6858x; 1.2305x over previous
import functools

import jax
import jax.numpy as jnp
from jax import lax
from jax.experimental import pallas as pl
from jax.experimental.pallas import tpu as pltpu
from jax.experimental.pallas import tpu_sc as plsc

B, C, L = 512, 64, 96
EMB = 16
H, F = 4, 4
HF = H * F
GH = 4
N = B * C
E = 1048576

NC, NS = 2, 16
NW = NC * NS
EPW = E // NW
K = 128
SRCW = 24
ERW = 8
ACCW = 24
NPT = N // NS


PB = 128
PN = PB * C


def _prep_body(feature, emb_W, emb_b, se_W1, se_b1, se_W2, se_b2, fc_W,
               AL, AR, src_tab, er_tab, m_ref):
    i = pl.program_id(0)
    x = feature[...].reshape(PN, L) @ emb_W[...] + emb_b[...]
    x3 = x.reshape(PB, C, EMB)
    s = jnp.mean(x3, axis=1)
    a = jax.nn.relu(s @ se_W1[...] + se_b1[...])
    g = jax.nn.sigmoid(a @ se_W2[...] + se_b2[...])
    embed = (x3 * g[:, None, :]).reshape(PN, EMB)
    feat = embed @ fc_W[...]
    el = feat @ AL[...]
    er = feat @ AR[...]
    mel = jnp.broadcast_to(jnp.max(el, axis=0)[:, None], (H, 16))
    mer = jnp.broadcast_to(jnp.max(er, axis=0)[:, None], (H, 16))
    mb = jnp.concatenate([mel, mer], axis=0)

    @pl.when(i == 0)
    def _():
        m_ref[...] = mb

    @pl.when(i > 0)
    def _():
        m_ref[...] = jnp.maximum(m_ref[...], mb)

    @pl.when(i == B // PB - 1)
    def _():
        m4 = m_ref[0:4, :] + m_ref[4:8, :]
        m_ref[0:4, :] = jnp.maximum(m4, 0.2 * m4)

    pad4 = jnp.zeros((PN, 4), jnp.float32)
    src_tab[...] = jnp.concatenate([el, feat, pad4], axis=1)
    er_tab[...] = jnp.concatenate([er, pad4], axis=1)


def _prep(feature, emb_W, emb_b, se_W1, se_b1, se_W2, se_b2, fc_W, AL, AR):
    grid = B // PB
    small = lambda shape: pl.BlockSpec(shape, lambda i: (0,) * len(shape))
    return pl.pallas_call(
        _prep_body,
        grid=(grid,),
        in_specs=[
            pl.BlockSpec((PB, C, L), lambda i: (i, 0, 0)),
            small((L, EMB)),
            small((1, EMB)),
            small((EMB, 4)),
            small((1, 4)),
            small((4, EMB)),
            small((1, EMB)),
            small((EMB, HF)),
            small((HF, H)),
            small((HF, H)),
        ],
        out_specs=[
            pl.BlockSpec((PN, SRCW), lambda i: (i, 0)),
            pl.BlockSpec((PN, ERW), lambda i: (i, 0)),
            pl.BlockSpec((8, 16), lambda i: (0, 0)),
        ],
        out_shape=[
            jax.ShapeDtypeStruct((N, SRCW), jnp.float32),
            jax.ShapeDtypeStruct((N, ERW), jnp.float32),
            jax.ShapeDtypeStruct((8, 16), jnp.float32),
        ],
    )(feature, emb_W, emb_b, se_W1, se_b1, se_W2, se_b2, fc_W, AL, AR)


CPB = 16
RPW = EPW // K
NB = RPW // CPB


def _edge_body(src_hbm, dst_hbm, src_tab, er_tab, m_hbm, zeros_hbm, out_hbm,
               sblk, dblk, sr0, erb0, sr1, erb1, sr2, erb2, sr3, erb3,
               ob0, ob1, m_v, acc, si, sg0, sg1, sg2, sg3, ss0, ss1):
    cid = lax.axis_index("c")
    sid = lax.axis_index("s")
    wid = cid * NS + sid

    srb = (sr0, sr1, sr2, sr3)
    erb = (erb0, erb1, erb2, erb3)
    ob = (ob0, ob1)
    sg = (sg0, sg1, sg2, sg3)
    ss = (ss0, ss1)

    pltpu.sync_copy(m_hbm.at[pl.ds(0, H)], m_v)
    pltpu.sync_copy(zeros_hbm, acc.at[pl.ds(sid * NPT, NPT)])
    plsc.subcore_barrier()

    ii = lax.iota(jnp.int32, 16)
    zero16 = jnp.zeros((16,), jnp.float32)
    for buf in ob:
        for j in range(K // 16):
            rows = ii + j * 16
            for c in range(4 + HF, ACCW):
                plsc.store_scatter(buf, [rows, jnp.full((16,), c, jnp.int32)],
                                   zero16)

    rbase = wid * RPW

    def load_block(t):
        half = (t % 2) * CPB
        pltpu.async_copy(src_hbm.at[pl.ds(rbase + t * CPB, CPB)],
                         sblk.at[pl.ds(half, CPB)], si)
        pltpu.async_copy(dst_hbm.at[pl.ds(rbase + t * CPB, CPB)],
                         dblk.at[pl.ds(half, CPB)], si)

    def wait_block():
        pltpu.make_async_copy(src_hbm.at[pl.ds(0, CPB)],
                              sblk.at[pl.ds(0, CPB)], si).wait()
        pltpu.make_async_copy(dst_hbm.at[pl.ds(0, CPB)],
                              dblk.at[pl.ds(0, CPB)], si).wait()

    def issue_gathers(ci, b):
        r = ci % (2 * CPB)
        pltpu.async_copy(src_tab.at[sblk.at[r]], srb[b], sg[b])
        pltpu.async_copy(er_tab.at[dblk.at[r]], erb[b], sg[b])

    def wait_gathers(b):
        pltpu.make_async_copy(src_tab.at[sblk.at[0]], srb[b], sg[b]).wait()
        pltpu.make_async_copy(er_tab.at[dblk.at[0]], erb[b], sg[b]).wait()

    def drain_scatter(b):
        pltpu.make_async_copy(ob[b], acc.at[dblk.at[0]], ss[b]).wait()

    load_block(0)
    wait_block()
    issue_gathers(0, 0)
    issue_gathers(1, 1)
    issue_gathers(2, 2)

    def outer(g, carry):
        for u in range(4):
            ci = g * 4 + u
            b = u % 2
            bg = u
            wait_gathers(bg)

            @pl.when((ci % CPB == 2) & (ci < (NB - 1) * CPB))
            def _():
                load_block(ci // CPB + 1)

            @pl.when((ci % CPB == CPB - 3) & (ci + 3 < RPW))
            def _():
                wait_block()

            @pl.when(ci + 3 < RPW)
            def _():
                issue_gathers(ci + 3, (u + 3) % 4)

            @pl.when(ci >= 2)
            def _():
                drain_scatter(b)

            for j in range(K // 16):
                rows = ii + j * 16
                chs = [jnp.full((16,), h, jnp.int32) for h in range(H)]
                els = [plsc.load_gather(srb[bg], [rows, chs[h]])
                       for h in range(H)]
                ers = [plsc.load_gather(erb[bg], [rows, chs[h]])
                      for h in range(H)]
                fvs = [plsc.load_gather(
                           srb[bg], [rows, jnp.full((16,), 4 + c, jnp.int32)])
                       for c in range(HF)]
                es = [els[h] + ers[h] for h in range(H)]
                es = [jnp.maximum(e, 0.2 * e) for e in es]
                ees = [jnp.exp(es[h] - m_v[h]) for h in range(H)]
                for h in range(H):
                    plsc.store_scatter(ob[b], [rows, chs[h]], ees[h])
                outs = [ees[c // F] * fvs[c] for c in range(HF)]
                for c in range(HF):
                    plsc.store_scatter(
                        ob[b], [rows, jnp.full((16,), 4 + c, jnp.int32)],
                        outs[c])
            pltpu.async_copy(ob[b], acc.at[dblk.at[ci % (2 * CPB)]], ss[b],
                             add=True)
        return carry

    lax.fori_loop(0, RPW // 4, outer, 0)
    drain_scatter(0)
    drain_scatter(1)

    plsc.subcore_barrier()
    pltpu.sync_copy(acc.at[pl.ds(sid * NPT, NPT)],
                    out_hbm.at[pl.ds(wid * NPT, NPT)])


def _edge(src2d, dst2d, src_tab, er_tab, m, zeros):
    mesh = plsc.VectorSubcoreMesh(core_axis_name="c", subcore_axis_name="s")
    f = functools.partial(
        pl.kernel,
        out_type=jax.ShapeDtypeStruct((NC * N, ACCW), jnp.float32),
        mesh=mesh,
        scratch_types=[
            pltpu.VMEM((2 * CPB, K), jnp.int32),
            pltpu.VMEM((2 * CPB, K), jnp.int32),
            pltpu.VMEM((K, SRCW), jnp.float32),
            pltpu.VMEM((K, ERW), jnp.float32),
            pltpu.VMEM((K, SRCW), jnp.float32),
            pltpu.VMEM((K, ERW), jnp.float32),
            pltpu.VMEM((K, SRCW), jnp.float32),
            pltpu.VMEM((K, ERW), jnp.float32),
            pltpu.VMEM((K, SRCW), jnp.float32),
            pltpu.VMEM((K, ERW), jnp.float32),
            pltpu.VMEM((K, ACCW), jnp.float32),
            pltpu.VMEM((K, ACCW), jnp.float32),
            pltpu.VMEM((H, 16), jnp.float32),
            pltpu.VMEM_SHARED((N, ACCW), jnp.float32),
            pltpu.SemaphoreType.DMA,
            pltpu.SemaphoreType.DMA,
            pltpu.SemaphoreType.DMA,
            pltpu.SemaphoreType.DMA,
            pltpu.SemaphoreType.DMA,
            pltpu.SemaphoreType.DMA,
            pltpu.SemaphoreType.DMA,
        ],
        compiler_params=pltpu.CompilerParams(needs_layout_passes=False,
                                             use_tc_tiling_on_sc=False),
    )(_edge_body)
    return f(src2d, dst2d, src_tab, er_tab, m, zeros)


GB = 64
GN = GB * C


def _post_body(p0, p1, gat_bias, proj_W, proj_b, gw1, gw2, w_ref, out_ref):
    accb = p0[...] + p1[...]
    den = accb[:, 0:4]
    msg = accb[:, 4:4 + HF]
    hrow = lax.broadcasted_iota(jnp.int32, (H, HF), 0)
    hcol = lax.broadcasted_iota(jnp.int32, (H, HF), 1) // F
    R = jnp.where(hrow == hcol, 1.0, 0.0)
    den16 = den @ R
    rst = jnp.where(den16 != 0.0, msg / den16, 0.0) + gat_bias[...]
    encode = rst.reshape(GB, C, HF)
    graph1 = lax.dot_general(encode, encode,
                             (((2,), (2,)), ((0,), (0,))))
    h = lax.dot_general(encode, proj_W[...],
                        (((2,), (0,)), ((), ()))) + proj_b[...][None]
    u = lax.dot_general(h, gw1[...], (((2,), (0,)), ((), ())))
    v = lax.dot_general(h, gw2[...], (((2,), (0,)), ((), ())))
    graph2 = u + v[:, :, 0][:, None, :]
    w = w_ref[...][None]
    out_ref[...] = w * graph1 + (1.0 - w) * graph2


def _post(partial, gat_bias, proj_W, proj_b, gw1, gw2, w_param):
    grid = B // GB
    small = lambda shape: pl.BlockSpec(shape, lambda i: (0,) * len(shape))
    return pl.pallas_call(
        _post_body,
        grid=(grid,),
        in_specs=[
            pl.BlockSpec((GN, ACCW), lambda i: (i, 0)),
            pl.BlockSpec((GN, ACCW), lambda i: (i + grid, 0)),
            small((1, HF)),
            small((EMB, GH)),
            small((1, GH)),
            small((GH, 1)),
            small((GH, 1)),
            small((C, C)),
        ],
        out_specs=pl.BlockSpec((GB, C, C), lambda i: (i, 0, 0)),
        out_shape=jax.ShapeDtypeStruct((B, C, C), jnp.float32),
    )(partial, partial, gat_bias, proj_W, proj_b, gw1, gw2, w_param)


def kernel(feature, edge_index, emb_W, emb_b, se_W1, se_b1, se_W2, se_b2,
           fc_W, attn_l, attn_r, gat_bias, proj_W, proj_b, graph_W, w_param):
    src = edge_index[0].reshape(E // K, K)
    dst = edge_index[1].reshape(E // K, K)
    eye = jnp.eye(H, dtype=jnp.float32)
    AL = (eye[:, None, :] * attn_l[:, :, None]).reshape(HF, H)
    AR = (eye[:, None, :] * attn_r[:, :, None]).reshape(HF, H)

    src_tab, er_tab, m = _prep(feature, emb_W, emb_b.reshape(1, EMB),
                               se_W1, se_b1.reshape(1, 4), se_W2,
                               se_b2.reshape(1, EMB), fc_W, AL, AR)
    zeros = jnp.zeros((NPT, ACCW), jnp.float32)
    partial = _edge(src, dst, src_tab, er_tab, m, zeros)
    return _post(partial, gat_bias.reshape(1, HF), proj_W,
                 proj_b.reshape(1, GH), graph_W[:GH], graph_W[GH:], w_param)
